# baseline (speedup 1.0000x reference)
import functools

import jax
import jax.numpy as jnp
from jax.experimental import pallas as pl

N = 48
NF = 23
NEF = 5
D = 64
HG = 64
HF = 128
HN = 128
HE = 128
JF = N * NEF
BBL = 128


def _fused(x_ref, adj_ref, wm1, ws1, b1, wm2, ws2, b2,
           wm3, ws3, b3, wf1, bf1, wf2, bf2, wn1, bn1, wn2n, bn2n, wn2m,
           we1, be1, wbs, node_out, adj_out, mask_out):
    f32 = jnp.float32
    dot = functools.partial(jnp.dot, preferred_element_type=f32)
    bdot = lambda a, b, dims: jax.lax.dot_general(
        a, b, dims, preferred_element_type=f32)

    a_fj = jnp.transpose(adj_ref[...].astype(jnp.bfloat16), (2, 0, 1))
    h2 = jnp.transpose(x_ref[...], (2, 1, 0)).reshape(BBL * N, NF)

    def gnn(h2, wm_ref, ws_ref, b_ref):
        dh = ws_ref.shape[1]
        parts = [dot(h2, wm_ref[f]).astype(jnp.bfloat16).reshape(BBL, N, dh)
                 for f in range(NEF)]
        hw = jnp.concatenate(parts, axis=1)
        msg = bdot(a_fj, hw, (((2,), (1,)), ((0,), (0,))))
        return jnp.maximum(
            msg.reshape(BBL * N, dh) + dot(h2, ws_ref[...]) + b_ref[...], 0.0)

    h2 = gnn(h2, wm1, ws1, b1)
    h2 = gnn(h2, wm2, ws2, b2)
    h2 = gnn(h2, wm3, ws3, b3)

    h2 = jnp.maximum(dot(h2, wf1[...]) + bf1[...], 0.0)
    ne = dot(h2, wf2[...]) + bf2[...]

    hn = jnp.maximum(dot(ne, wn1[...]) + bn1[...], 0.0)
    na = (dot(hn, wn2n[...]) + bn2n[...]).reshape(BBL, N, NF)
    node_out[...] = jnp.transpose(na, (2, 1, 0))
    mask_out[...] = dot(hn, wn2m[...]).reshape(BBL, N, 1)

    p = jnp.maximum(dot(ne, we1[...]) + be1[...], 0.0)
    tparts = [dot(p, wbs[f]).astype(jnp.bfloat16).reshape(BBL, N, HE)
              for f in range(NEF)]
    t_fj = jnp.concatenate(tparts, axis=1)
    out_fj = bdot(p.astype(jnp.bfloat16).reshape(BBL, N, HE), t_fj,
                  (((2,), (2,)), ((0,), (0,))))
    adj_out[...] = jnp.transpose(out_fj, (1, 2, 0))


@jax.jit
def kernel(x, adj, W_msg1, W_self1, b1, W_msg2, W_self2, b2, W_msg3, W_self3,
           b3, Wf1, bf1, Wf2, bf2, Wn1, bn1, Wn2, bn2, We1, be1, Wb):
    B = x.shape[0]
    adj_v = adj.transpose(1, 3, 2, 0).reshape(N, JF, B)
    x_v = x.transpose(2, 1, 0)
    wm1 = W_msg1.reshape(NEF, NF, HG)
    wm2 = W_msg2.reshape(NEF, HG, HG)
    wm3 = W_msg3.reshape(NEF, HG, HG)
    wbs = 0.5 * (Wb + Wb.transpose(0, 2, 1))
    row = lambda v: v.reshape(1, -1)

    grid = (B // BBL,)
    lanes = lambda *shape: pl.BlockSpec(
        shape, lambda i: (0,) * (len(shape) - 1) + (i,))
    wspec = lambda w: pl.BlockSpec(w.shape, lambda i: (0,) * w.ndim)

    weights = (wm1, W_self1, row(b1), wm2, W_self2, row(b2), wm3,
               W_self3, row(b3), Wf1, row(bf1), Wf2, row(bf2), Wn1, row(bn1),
               Wn2[:, 1:], row(bn2[1:]), Wn2[:, :1], We1, row(be1), wbs)
    mask_bias = bn2[0]

    node_v, out_v, mask3 = pl.pallas_call(
        _fused,
        grid=grid,
        in_specs=[lanes(NF, N, BBL), lanes(N, JF, BBL)] +
                 [wspec(w) for w in weights],
        out_specs=[lanes(NF, N, BBL), lanes(N, JF, BBL),
                   pl.BlockSpec((BBL, N, 1), lambda i: (i, 0, 0))],
        out_shape=[
            jax.ShapeDtypeStruct((NF, N, B), jnp.float32),
            jax.ShapeDtypeStruct((N, JF, B), jnp.float32),
            jax.ShapeDtypeStruct((B, N, 1), jnp.float32),
        ],
    )(x_v, adj_v, *weights)

    node_logits = node_v.transpose(2, 1, 0)
    adj_logits = out_v.reshape(N, NEF, N, B).transpose(3, 0, 2, 1)
    mask_logits = mask3.reshape(B, N) + mask_bias
    return node_logits, adj_logits, mask_logits


# speedup vs baseline: 1.0008x; 1.0008x over previous
import functools

import jax
import jax.numpy as jnp
from jax.experimental import pallas as pl

N = 48
NF = 23
NEF = 5
D = 64
HG = 64
HF = 128
HN = 128
HE = 128
JF = N * NEF
BBL = 128


def _fused(x_ref, adj_ref, wm1, ws1, b1, wm2, ws2, b2,
           wm3, ws3, b3, wf1, bf1, wf2, bf2, wn1, bn1, wn2n, bn2n, wn2m,
           we1, be1, wbs, node_out, adj_out, mask_out):
    f32 = jnp.float32
    dot = functools.partial(jnp.dot, preferred_element_type=f32)
    bdot = lambda a, b, dims: jax.lax.dot_general(
        a, b, dims, preferred_element_type=f32)

    a_fj = jnp.transpose(adj_ref[...], (2, 0, 1))
    h2 = jnp.transpose(x_ref[...], (2, 1, 0)).reshape(BBL * N, NF)

    def gnn(h2, wm_ref, ws_ref, b_ref):
        dh = ws_ref.shape[1]
        parts = [dot(h2, wm_ref[f]).reshape(BBL, N, dh) for f in range(NEF)]
        hw = jnp.concatenate(parts, axis=1)
        msg = bdot(a_fj, hw, (((2,), (1,)), ((0,), (0,))))
        return jnp.maximum(
            msg.reshape(BBL * N, dh) + dot(h2, ws_ref[...]) + b_ref[...], 0.0)

    h2 = gnn(h2, wm1, ws1, b1)
    h2 = gnn(h2, wm2, ws2, b2)
    h2 = gnn(h2, wm3, ws3, b3)

    h2 = jnp.maximum(dot(h2, wf1[...]) + bf1[...], 0.0)
    ne = dot(h2, wf2[...]) + bf2[...]

    hn = jnp.maximum(dot(ne, wn1[...]) + bn1[...], 0.0)
    na = (dot(hn, wn2n[...]) + bn2n[...]).reshape(BBL, N, NF)
    node_out[...] = jnp.transpose(na, (2, 1, 0))
    mask_out[...] = dot(hn, wn2m[...]).reshape(BBL, N, 1)

    p = jnp.maximum(dot(ne, we1[...]) + be1[...], 0.0)
    tparts = [dot(p, wbs[f]).reshape(BBL, N, HE) for f in range(NEF)]
    t_fj = jnp.concatenate(tparts, axis=1)
    out_fj = bdot(p.reshape(BBL, N, HE), t_fj,
                  (((2,), (2,)), ((0,), (0,))))
    adj_out[...] = jnp.transpose(out_fj, (1, 2, 0))


@jax.jit
def kernel(x, adj, W_msg1, W_self1, b1, W_msg2, W_self2, b2, W_msg3, W_self3,
           b3, Wf1, bf1, Wf2, bf2, Wn1, bn1, Wn2, bn2, We1, be1, Wb):
    B = x.shape[0]
    adj_v = adj.transpose(1, 3, 2, 0).reshape(N, JF, B)
    x_v = x.transpose(2, 1, 0)
    wm1 = W_msg1.reshape(NEF, NF, HG)
    wm2 = W_msg2.reshape(NEF, HG, HG)
    wm3 = W_msg3.reshape(NEF, HG, HG)
    wbs = 0.5 * (Wb + Wb.transpose(0, 2, 1))
    row = lambda v: v.reshape(1, -1)

    grid = (B // BBL,)
    lanes = lambda *shape: pl.BlockSpec(
        shape, lambda i: (0,) * (len(shape) - 1) + (i,))
    wspec = lambda w: pl.BlockSpec(w.shape, lambda i: (0,) * w.ndim)

    weights = (wm1, W_self1, row(b1), wm2, W_self2, row(b2), wm3,
               W_self3, row(b3), Wf1, row(bf1), Wf2, row(bf2), Wn1, row(bn1),
               Wn2[:, 1:], row(bn2[1:]), Wn2[:, :1], We1, row(be1), wbs)
    mask_bias = bn2[0]

    node_v, out_v, mask3 = pl.pallas_call(
        _fused,
        grid=grid,
        in_specs=[lanes(NF, N, BBL), lanes(N, JF, BBL)] +
                 [wspec(w) for w in weights],
        out_specs=[lanes(NF, N, BBL), lanes(N, JF, BBL),
                   pl.BlockSpec((BBL, N, 1), lambda i: (i, 0, 0))],
        out_shape=[
            jax.ShapeDtypeStruct((NF, N, B), jnp.float32),
            jax.ShapeDtypeStruct((N, JF, B), jnp.float32),
            jax.ShapeDtypeStruct((B, N, 1), jnp.float32),
        ],
    )(x_v, adj_v, *weights)

    node_logits = node_v.transpose(2, 1, 0)
    adj_logits = out_v.reshape(N, NEF, N, B).transpose(3, 0, 2, 1)
    mask_logits = mask3.reshape(B, N) + mask_bias
    return node_logits, adj_logits, mask_logits


# speedup vs baseline: 1.0088x; 1.0080x over previous
import functools

import jax
import jax.numpy as jnp
from jax.experimental import pallas as pl

N = 48
NF = 23
NEF = 5
D = 64
HG = 64
HF = 128
HN = 128
HE = 128
JF = N * NEF
BBL = 128
CH = 64


def _fused(x_ref, adj_ref, wm1, ws1, b1, wm2, ws2, b2,
           wm3, ws3, b3, wf1, bf1, wf2, bf2, wn1, bn1, wn2n, bn2n, wn2m,
           we1, be1, wbs, node_out, adj_out, mask_out):
    f32 = jnp.float32
    dot = functools.partial(jnp.dot, preferred_element_type=f32)
    bdot = lambda a, b, dims: jax.lax.dot_general(
        a, b, dims, preferred_element_type=f32)

    for c in range(BBL // CH):
        sl = pl.ds(c * CH, CH)
        a_fj = jnp.transpose(adj_ref[:, :, sl], (2, 0, 1))
        h2 = jnp.transpose(x_ref[:, :, sl], (2, 1, 0)).reshape(CH * N, NF)

        def gnn(h2, wm_ref, ws_ref, b_ref):
            dh = ws_ref.shape[1]
            parts = [dot(h2, wm_ref[f]).reshape(CH, N, dh)
                     for f in range(NEF)]
            hw = jnp.concatenate(parts, axis=1)
            msg = bdot(a_fj, hw, (((2,), (1,)), ((0,), (0,))))
            return jnp.maximum(
                msg.reshape(CH * N, dh) + dot(h2, ws_ref[...]) + b_ref[...],
                0.0)

        h2 = gnn(h2, wm1, ws1, b1)
        h2 = gnn(h2, wm2, ws2, b2)
        h2 = gnn(h2, wm3, ws3, b3)

        h2 = jnp.maximum(dot(h2, wf1[...]) + bf1[...], 0.0)
        ne = dot(h2, wf2[...]) + bf2[...]

        hn = jnp.maximum(dot(ne, wn1[...]) + bn1[...], 0.0)
        na = (dot(hn, wn2n[...]) + bn2n[...]).reshape(CH, N, NF)
        node_out[:, :, sl] = jnp.transpose(na, (2, 1, 0))
        mask_out[sl] = dot(hn, wn2m[...]).reshape(CH, N, 1)

        p = jnp.maximum(dot(ne, we1[...]) + be1[...], 0.0)
        tparts = [dot(p, wbs[f]).reshape(CH, N, HE) for f in range(NEF)]
        t_fj = jnp.concatenate(tparts, axis=1)
        out_fj = bdot(p.reshape(CH, N, HE), t_fj,
                      (((2,), (2,)), ((0,), (0,))))
        adj_out[:, :, sl] = jnp.transpose(out_fj, (1, 2, 0))


@jax.jit
def kernel(x, adj, W_msg1, W_self1, b1, W_msg2, W_self2, b2, W_msg3, W_self3,
           b3, Wf1, bf1, Wf2, bf2, Wn1, bn1, Wn2, bn2, We1, be1, Wb):
    B = x.shape[0]
    adj_v = adj.transpose(1, 3, 2, 0).reshape(N, JF, B)
    x_v = x.transpose(2, 1, 0)
    wm1 = W_msg1.reshape(NEF, NF, HG)
    wm2 = W_msg2.reshape(NEF, HG, HG)
    wm3 = W_msg3.reshape(NEF, HG, HG)
    wbs = 0.5 * (Wb + Wb.transpose(0, 2, 1))
    row = lambda v: v.reshape(1, -1)

    grid = (B // BBL,)
    lanes = lambda *shape: pl.BlockSpec(
        shape, lambda i: (0,) * (len(shape) - 1) + (i,))
    wspec = lambda w: pl.BlockSpec(w.shape, lambda i: (0,) * w.ndim)

    weights = (wm1, W_self1, row(b1), wm2, W_self2, row(b2), wm3,
               W_self3, row(b3), Wf1, row(bf1), Wf2, row(bf2), Wn1, row(bn1),
               Wn2[:, 1:], row(bn2[1:]), Wn2[:, :1], We1, row(be1), wbs)
    mask_bias = bn2[0]

    node_v, out_v, mask3 = pl.pallas_call(
        _fused,
        grid=grid,
        in_specs=[lanes(NF, N, BBL), lanes(N, JF, BBL)] +
                 [wspec(w) for w in weights],
        out_specs=[lanes(NF, N, BBL), lanes(N, JF, BBL),
                   pl.BlockSpec((BBL, N, 1), lambda i: (i, 0, 0))],
        out_shape=[
            jax.ShapeDtypeStruct((NF, N, B), jnp.float32),
            jax.ShapeDtypeStruct((N, JF, B), jnp.float32),
            jax.ShapeDtypeStruct((B, N, 1), jnp.float32),
        ],
    )(x_v, adj_v, *weights)

    node_logits = node_v.transpose(2, 1, 0)
    adj_logits = out_v.reshape(N, NEF, N, B).transpose(3, 0, 2, 1)
    mask_logits = mask3.reshape(B, N) + mask_bias
    return node_logits, adj_logits, mask_logits
